# Initial kernel scaffold; baseline (speedup 1.0000x reference)
import jax
import jax.numpy as jnp
from jax.experimental import pallas as pl


def kernel(x, edge_index, pos_edge_index, neg_edge_index, temp, W0, b0, W1, b1, Wm, bm, Wl, bl, Wq, bq, Wd1, bd1, WdX, bdX, Wa, ba):
    raise NotImplementedError("write your pallas kernel here")


# speedup vs baseline: 1.4880x; 1.4880x over previous
import functools

import jax
import jax.numpy as jnp
from jax import lax
from jax.experimental import pallas as pl
from jax.experimental.pallas import tpu as pltpu
from jax.experimental.pallas import tpu_sc as plsc

_N = 10000
_NPAD = 10240
_E = 320000
_EPAD = 327680
_EP = 100000
_EPPAD = 102400
_D = 128
_H2 = 256
_CAT = 8
_EPSV = 1e-07

_NC = 2
_NS = 16
_NW = _NC * _NS

_C = 128

_f32 = jnp.float32


def _deg_body(idx_hbm, out_hbm, idxv, ones_v, cnt_sh, sem):
    c = lax.axis_index("c")
    s = lax.axis_index("s")

    def fill(i, v):
        for k in range(8):
            ones_v[i, pl.ds(k * 16, 16)] = jnp.full((16,), v, _f32)
        return v

    lax.fori_loop(0, _C, fill, 0.0)

    def zero_chunk(k, _):
        pltpu.sync_copy(ones_v, cnt_sh.at[pl.ds(s * 640 + k * _C, _C)])
        return _

    lax.fori_loop(0, 5, zero_chunk, 0)
    lax.fori_loop(0, _C, fill, 1.0)
    plsc.subcore_barrier()

    def chunk(j, _):
        base = c * _EPAD + s * 20480 + j * _C
        pltpu.sync_copy(idx_hbm.at[pl.ds(base, _C)], idxv)
        pltpu.sync_copy(ones_v, cnt_sh.at[idxv], add=True)
        return _

    lax.fori_loop(0, 160, chunk, 0)
    plsc.subcore_barrier()

    def out_chunk(k, _):
        off = s * 640 + k * _C
        pltpu.sync_copy(cnt_sh.at[pl.ds(off, _C)],
                        out_hbm.at[pl.ds(c * _NPAD + off, _C)])
        return _

    lax.fori_loop(0, 5, out_chunk, 0)


def _deg_call(idx_all):
    mesh = plsc.VectorSubcoreMesh(core_axis_name="c", subcore_axis_name="s")
    return pl.kernel(
        _deg_body,
        out_type=jax.ShapeDtypeStruct((2 * _NPAD, _D), _f32),
        mesh=mesh,
        scratch_types=[
            pltpu.VMEM((_C,), jnp.int32),
            pltpu.VMEM((_C, _D), _f32),
            pltpu.VMEM_SHARED((_NPAD, _D), _f32),
            pltpu.SemaphoreType.DMA,
        ],
    )(idx_all)


def _conv_body(xn_hbm, src_hbm, dst_hbm, out_hbm, srcv, dstv, rows, agg_sh, sem):
    c = lax.axis_index("c")
    s = lax.axis_index("s")
    wid = s * _NC + c

    def zrow(i, _):
        for k in range(8):
            rows[i, pl.ds(k * 16, 16)] = jnp.zeros((16,), _f32)
        return _

    lax.fori_loop(0, _C, zrow, 0)

    def zero_chunk(k, _):
        pltpu.sync_copy(rows, agg_sh.at[pl.ds(s * 640 + k * _C, _C)])
        return _

    lax.fori_loop(0, 5, zero_chunk, 0)
    plsc.subcore_barrier()

    def chunk(j, _):
        base = wid * 10240 + j * _C
        pltpu.sync_copy(src_hbm.at[pl.ds(base, _C)], srcv)
        pltpu.sync_copy(dst_hbm.at[pl.ds(base, _C)], dstv)
        pltpu.async_copy(xn_hbm.at[srcv], rows, sem).wait()
        pltpu.sync_copy(rows, agg_sh.at[dstv], add=True)
        return _

    lax.fori_loop(0, 80, chunk, 0)
    plsc.subcore_barrier()

    def out_chunk(k, _):
        off = s * 640 + k * _C
        pltpu.sync_copy(agg_sh.at[pl.ds(off, _C)],
                        out_hbm.at[pl.ds(c * _NPAD + off, _C)])
        return _

    lax.fori_loop(0, 5, out_chunk, 0)


def _conv_call(xn_pad, src_pad, dst_pad):
    mesh = plsc.VectorSubcoreMesh(core_axis_name="c", subcore_axis_name="s")
    return pl.kernel(
        _conv_body,
        out_type=jax.ShapeDtypeStruct((2 * _NPAD, _D), _f32),
        mesh=mesh,
        scratch_types=[
            pltpu.VMEM((_C,), jnp.int32),
            pltpu.VMEM((_C,), jnp.int32),
            pltpu.VMEM((_C, _D), _f32),
            pltpu.VMEM_SHARED((_NPAD, _D), _f32),
            pltpu.SemaphoreType.DMA,
        ],
    )(xn_pad, src_pad, dst_pad)


def _pair_body(h_hbm, p0_hbm, p1_hbm, n0_hbm, n1_hbm, oP, oN,
               ia, ib, ra, rb, semA, semB):
    c = lax.axis_index("c")
    s = lax.axis_index("s")
    wid = s * _NC + c

    def run(a_hbm, b_hbm, out_hbm, t):
        def chunk(j, _):
            base = t * 6400 + j * _C
            pltpu.sync_copy(a_hbm.at[pl.ds(base, _C)], ia)
            pltpu.sync_copy(b_hbm.at[pl.ds(base, _C)], ib)
            da = pltpu.async_copy(h_hbm.at[ia], ra, semA)
            db = pltpu.async_copy(h_hbm.at[ib], rb, semB)
            da.wait()
            db.wait()

            def addrow(i, _2):
                for k in range(8):
                    sl = pl.ds(k * 16, 16)
                    ra[i, sl] = ra[i, sl] + rb[i, sl]
                return _2

            lax.fori_loop(0, _C, addrow, 0)
            pltpu.sync_copy(ra, out_hbm.at[pl.ds(base, _C)])
            return _

        lax.fori_loop(0, 50, chunk, 0)

    @pl.when(wid < 16)
    def _():
        run(p0_hbm, p1_hbm, oP, wid)

    @pl.when(wid >= 16)
    def _():
        run(n0_hbm, n1_hbm, oN, wid - 16)


def _pair_call(h2_pad, p0, p1, n0, n1):
    mesh = plsc.VectorSubcoreMesh(core_axis_name="c", subcore_axis_name="s")
    return pl.kernel(
        _pair_body,
        out_type=(jax.ShapeDtypeStruct((_EPPAD, _D), _f32),
                  jax.ShapeDtypeStruct((_EPPAD, _D), _f32)),
        mesh=mesh,
        scratch_types=[
            pltpu.VMEM((_C,), jnp.int32),
            pltpu.VMEM((_C,), jnp.int32),
            pltpu.VMEM((_C, _D), _f32),
            pltpu.VMEM((_C, _D), _f32),
            pltpu.SemaphoreType.DMA,
            pltpu.SemaphoreType.DMA,
        ],
    )(h2_pad, p0, p1, n0, n1)


def _mm_body(x_ref, w_ref, b_ref, o_ref, *, relu):
    y = jnp.dot(x_ref[...], w_ref[...], preferred_element_type=_f32)
    y = y + b_ref[...]
    o_ref[...] = jnp.maximum(y, 0.0) if relu else y


def _mm(xmat, W, b, relu):
    rows = xmat.shape[0]
    blk = 512
    return pl.pallas_call(
        functools.partial(_mm_body, relu=relu),
        grid=(rows // blk,),
        in_specs=[
            pl.BlockSpec((blk, _D), lambda r: (r, 0)),
            pl.BlockSpec((_D, _D), lambda r: (0, 0)),
            pl.BlockSpec((1, _D), lambda r: (0, 0)),
        ],
        out_specs=pl.BlockSpec((blk, _D), lambda r: (r, 0)),
        out_shape=jax.ShapeDtypeStruct((rows, _D), _f32),
    )(xmat, W, b.reshape(1, _D))


def _branch_body(tinv_ref, np_ref, noise_ref, u_ref, Wm_ref, bm_ref,
                 Wl_ref, bl_ref, Wq_ref, bq_ref, Rep_ref, F_ref,
                 Wd1_ref, bd1_ref, WdX_ref, bdX_ref, Wa_ref, ba_ref,
                 mean_o, logstd_o, q_o, A_o, X_o):
    npb = np_ref[...]
    mean = jnp.dot(npb, Wm_ref[...], preferred_element_type=_f32) + bm_ref[...]
    mean_o[...] = mean
    logstd = jnp.dot(npb, Wl_ref[...], preferred_element_type=_f32) + bl_ref[...]
    logstd_o[...] = logstd
    q = jnp.dot(npb, Wq_ref[...], preferred_element_type=_f32) + bq_ref[...]
    q_o[...] = q

    u = u_ref[...]
    g = -jnp.log(-jnp.log(u + _EPSV) + _EPSV)
    col = lax.broadcasted_iota(jnp.int32, q.shape, 1)
    logits = (q + g) * tinv_ref[0, 0]
    logits = jnp.where(col < _CAT, logits, -1e30)
    logits = logits - jnp.max(logits, axis=-1, keepdims=True)
    ez = jnp.exp(logits)
    z = ez / jnp.sum(ez, axis=-1, keepdims=True)

    nz = noise_ref[...] * jnp.exp(logstd) + mean
    zbig = jnp.dot(z, Rep_ref[...], preferred_element_type=_f32)
    m = jnp.dot(zbig * nz, F_ref[...], preferred_element_type=_f32)

    x1 = jnp.dot(m, Wd1_ref[...], preferred_element_type=_f32) + bd1_ref[...]
    x1 = jnp.maximum(x1, 0.0)
    x2 = jnp.dot(x1, WdX_ref[...], preferred_element_type=_f32) + bdX_ref[...]
    X_o[...] = jnp.maximum(x2, 0.0)

    al = jnp.dot(m, Wa_ref[...], preferred_element_type=_f32) + ba_ref[...]
    al = al - jnp.max(al, axis=-1, keepdims=True)
    ea = jnp.exp(al)
    A_o[...] = ea / jnp.sum(ea, axis=-1, keepdims=True)


def _branch_call(tinv, npemb_pad, noise, u_pad, weights):
    blk = 400
    grid = (_EP // blk,)
    full = lambda shp: pl.BlockSpec(shp, lambda r: (0, 0))
    return pl.pallas_call(
        _branch_body,
        grid=grid,
        in_specs=[
            pl.BlockSpec(memory_space=pltpu.SMEM),
            pl.BlockSpec((blk, _D), lambda r: (r, 0)),
            pl.BlockSpec((blk, _H2), lambda r: (r, 0)),
            pl.BlockSpec((blk, _D), lambda r: (r, 0)),
            full((_D, _H2)), full((1, _H2)),
            full((_D, _H2)), full((1, _H2)),
            full((_D, _D)), full((1, _D)),
            full((_D, _H2)),
            full((_H2, _D)),
            full((_D, _D)), full((1, _D)),
            full((_D, _D)), full((1, _D)),
            full((_D, _D)), full((1, _D)),
        ],
        out_specs=[
            pl.BlockSpec((blk, _H2), lambda r: (r, 0)),
            pl.BlockSpec((blk, _H2), lambda r: (r, 0)),
            pl.BlockSpec((blk, _D), lambda r: (r, 0)),
            pl.BlockSpec((blk, _D), lambda r: (r, 0)),
            pl.BlockSpec((blk, _D), lambda r: (r, 0)),
        ],
        out_shape=[
            jax.ShapeDtypeStruct((_EP, _H2), _f32),
            jax.ShapeDtypeStruct((_EP, _H2), _f32),
            jax.ShapeDtypeStruct((_EP, _D), _f32),
            jax.ShapeDtypeStruct((_EP, _D), _f32),
            jax.ShapeDtypeStruct((_EP, _D), _f32),
        ],
    )(tinv, npemb_pad, noise, u_pad, *weights)


_USE_SC_DEG = True
_USE_SC_CONV = True
_USE_SC_PAIR = True


def _deg_spec(idx_all):
    co = jnp.zeros((_NPAD,), _f32).at[idx_all[:_EPAD]].add(1.0)
    ci = jnp.zeros((_NPAD,), _f32).at[idx_all[_EPAD:]].add(1.0)
    return jnp.concatenate([co, ci])[:, None] * jnp.ones((1, _D), _f32)


def _conv_spec(xn_pad, src_pad, dst_pad):
    agg = jax.ops.segment_sum(xn_pad[src_pad], dst_pad, num_segments=_NPAD)
    return jnp.zeros((2 * _NPAD, _D), _f32).at[:_NPAD].set(agg)


def _pair_spec(h2, p0, p1, n0, n1):
    return h2[p0] + h2[p1], h2[n0] + h2[n1]


def kernel(x, edge_index, pos_edge_index, neg_edge_index, temp,
           W0, b0, W1, b1, Wm, bm, Wl, bl, Wq, bq,
           Wd1, bd1, WdX, bdX, Wa, ba):
    ei_pad = jnp.pad(edge_index, ((0, 0), (0, _EPAD - _E)),
                     constant_values=_N)
    src_pad = ei_pad[0]
    dst_pad = ei_pad[1]

    idx_all = jnp.concatenate([src_pad, dst_pad])
    cnt = (_deg_call if _USE_SC_DEG else _deg_spec)(idx_all)
    deg_out = cnt[:_N, 0]
    deg_in = cnt[_NPAD:_NPAD + _N, 0]
    norm_out = jnp.where(deg_out > 0, deg_out ** -0.5, 0.0)
    norm_in = jnp.where(deg_in > 0, deg_in ** -0.5, 0.0)
    norm_out_pad = jnp.pad(norm_out, (0, _NPAD - _N))[:, None]
    norm_in_pad = jnp.pad(norm_in, (0, _NPAD - _N))[:, None]

    xn = jnp.pad(x * norm_out[:, None], ((0, _NPAD - _N), (0, 0)))
    aggp = (_conv_call if _USE_SC_CONV else _conv_spec)(xn, src_pad, dst_pad)
    agg = (aggp[:_NPAD] + aggp[_NPAD:]) * norm_in_pad
    h = _mm(agg, W0, b0, relu=True)
    hn = h * norm_out_pad

    aggp2 = (_conv_call if _USE_SC_CONV else _conv_spec)(hn, src_pad, dst_pad)
    agg2 = (aggp2[:_NPAD] + aggp2[_NPAD:]) * norm_in_pad
    h2 = _mm(agg2, W1, b1, relu=False)

    posi = jnp.pad(pos_edge_index, ((0, 0), (0, _EPPAD - _EP)))
    negi = jnp.pad(neg_edge_index, ((0, 0), (0, _EPPAD - _EP)))
    npP, npN = (_pair_call if _USE_SC_PAIR else _pair_spec)(h2, posi[0], posi[1], negi[0], negi[1])

    noiseP = jax.random.normal(jax.random.key(42), (_EP, _H2), _f32)
    uP = jax.random.uniform(jax.random.key(43), (_EP, _CAT), _f32)
    noiseN = jax.random.normal(jax.random.key(44), (_EP, _H2), _f32)
    uN = jax.random.uniform(jax.random.key(45), (_EP, _CAT), _f32)
    uP_pad = jnp.pad(uP, ((0, 0), (0, _D - _CAT)), constant_values=0.5)
    uN_pad = jnp.pad(uN, ((0, 0), (0, _D - _CAT)), constant_values=0.5)

    Wqp = jnp.zeros((_D, _D), _f32).at[:, :_CAT].set(Wq)
    bqp = jnp.zeros((1, _D), _f32).at[0, :_CAT].set(bq)
    r_ = jnp.arange(_D)[:, None]
    c_ = jnp.arange(_H2)[None, :]
    Rep = (c_ // 32 == r_).astype(_f32)
    i_ = jnp.arange(_H2)[:, None]
    j_ = jnp.arange(_D)[None, :]
    Fm = (i_ % 32 == j_).astype(_f32)
    Wd1p = jnp.zeros((_D, _D), _f32).at[:32, :64].set(Wd1)
    bd1p = jnp.zeros((1, _D), _f32).at[0, :64].set(bd1)
    WdXp = jnp.zeros((_D, _D), _f32).at[:64, :].set(WdX)
    bdXp = bdX.reshape(1, _D)
    Wap = jnp.zeros((_D, _D), _f32).at[:32, :_CAT].set(Wa)
    bap = jnp.full((1, _D), -1e30, _f32).at[0, :_CAT].set(ba)

    weights = (Wm, bm.reshape(1, _H2), Wl, bl.reshape(1, _H2),
               Wqp, bqp, Rep, Fm, Wd1p, bd1p, WdXp, bdXp, Wap, bap)
    tinv = (1.0 / jnp.asarray(temp, _f32)).reshape(1, 1)

    pos_mean, pos_logstd, posq, posA, posX = _branch_call(
        tinv, npP, noiseP, uP_pad, weights)
    neg_mean, neg_logstd, negq, negA, negX = _branch_call(
        tinv, npN, noiseN, uN_pad, weights)

    return (posA[:, :_CAT], negA[:, :_CAT], posX, negX,
            pos_mean, neg_mean, pos_logstd, neg_logstd,
            posq[:, :_CAT], negq[:, :_CAT])


# speedup vs baseline: 1.6068x; 1.0798x over previous
import functools

import jax
import jax.numpy as jnp
from jax import lax
from jax.experimental import pallas as pl
from jax.experimental.pallas import tpu as pltpu
from jax.experimental.pallas import tpu_sc as plsc

_N = 10000
_NPAD = 10240
_E = 320000
_EPAD = 327680
_EP = 100000
_EPPAD = 102400
_D = 128
_H2 = 256
_CAT = 8
_EPSV = 1e-07

_NC = 2
_NS = 16
_NW = _NC * _NS

_C = 128

_f32 = jnp.float32


def _deg_body(idxm_hbm, out_hbm, idxv, ones_v, cnt_sh, sem, semw):
    c = lax.axis_index("c")
    s = lax.axis_index("s")

    def fill(i, v):
        for k in range(8):
            ones_v[i, pl.ds(k * 16, 16)] = jnp.full((16,), v, _f32)
        return v

    lax.fori_loop(0, _C, fill, 0.0)

    def zero_chunk(k, _):
        pltpu.sync_copy(ones_v, cnt_sh.at[pl.ds(s * 640 + k * _C, _C)])
        return _

    lax.fori_loop(0, 5, zero_chunk, 0)
    lax.fori_loop(0, _C, fill, 1.0)

    pltpu.sync_copy(idxm_hbm.at[pl.ds(c * 2560 + s * 160, 160)], idxv)
    plsc.subcore_barrier()

    def grp(g, _):
        for k in range(8):
            pltpu.async_copy(ones_v, cnt_sh.at[idxv.at[g * 8 + k]], sem,
                             add=True)
        for k in range(8):
            pltpu.make_async_copy(ones_v, cnt_sh.at[idxv.at[0]], sem).wait()
        return _

    lax.fori_loop(0, 20, grp, 0)
    plsc.subcore_barrier()

    for k in range(5):
        off = s * 640 + k * _C
        pltpu.async_copy(cnt_sh.at[pl.ds(off, _C)],
                         out_hbm.at[pl.ds(c * _NPAD + off, _C)], semw)
    for k in range(5):
        pltpu.make_async_copy(cnt_sh.at[pl.ds(0, _C)],
                              out_hbm.at[pl.ds(0, _C)], semw).wait()


def _deg_call(idx_all):
    mesh = plsc.VectorSubcoreMesh(core_axis_name="c", subcore_axis_name="s")
    return pl.kernel(
        _deg_body,
        out_type=jax.ShapeDtypeStruct((2 * _NPAD, _D), _f32),
        mesh=mesh,
        scratch_types=[
            pltpu.VMEM((160, _C), jnp.int32),
            pltpu.VMEM((_C, _D), _f32),
            pltpu.VMEM_SHARED((_NPAD, _D), _f32),
            pltpu.SemaphoreType.DMA,
            pltpu.SemaphoreType.DMA,
        ],
    )(idx_all.reshape(5120, _C))


def _conv_body(xn_hbm, srcm_hbm, dstm_hbm, out_hbm,
               iaA, iaB, didx, rowsA, rowsB, agg_sh,
               semIA, semIB, semGA, semGB, semSA, semSB, semw):
    c = lax.axis_index("c")
    s = lax.axis_index("s")
    wid = s * _NC + c

    def zrow(i, _):
        for k in range(8):
            rowsA[i, pl.ds(k * 16, 16)] = jnp.zeros((16,), _f32)
        return _

    lax.fori_loop(0, _C, zrow, 0)

    def zero_chunk(k, _):
        pltpu.sync_copy(rowsA, agg_sh.at[pl.ds(s * 640 + k * _C, _C)])
        return _

    lax.fori_loop(0, 5, zero_chunk, 0)

    pltpu.sync_copy(dstm_hbm.at[pl.ds(wid * 80, 80)], didx)
    plsc.subcore_barrier()

    pltpu.async_copy(srcm_hbm.at[wid * 80], iaA, semIA)
    pltpu.async_copy(srcm_hbm.at[wid * 80 + 1], iaB, semIB)

    def grp(g, carry):
        pltpu.make_async_copy(srcm_hbm.at[0], iaA, semIA).wait()
        pltpu.async_copy(xn_hbm.at[iaA], rowsA, semGA)
        pltpu.make_async_copy(srcm_hbm.at[0], iaB, semIB).wait()
        pltpu.async_copy(xn_hbm.at[iaB], rowsB, semGB)

        pltpu.make_async_copy(xn_hbm.at[iaA], rowsA, semGA).wait()
        pltpu.async_copy(rowsA, agg_sh.at[didx.at[2 * g]], semSA, add=True)

        @pl.when(g < 39)
        def _pfA():
            pltpu.async_copy(srcm_hbm.at[wid * 80 + 2 * g + 2], iaA, semIA)

        pltpu.make_async_copy(xn_hbm.at[iaB], rowsB, semGB).wait()
        pltpu.async_copy(rowsB, agg_sh.at[didx.at[2 * g + 1]], semSB, add=True)

        @pl.when(g < 39)
        def _pfB():
            pltpu.async_copy(srcm_hbm.at[wid * 80 + 2 * g + 3], iaB, semIB)

        pltpu.make_async_copy(rowsA, agg_sh.at[didx.at[0]], semSA).wait()
        pltpu.make_async_copy(rowsB, agg_sh.at[didx.at[0]], semSB).wait()
        return carry

    lax.fori_loop(0, 40, grp, 0)
    plsc.subcore_barrier()

    for k in range(5):
        off = s * 640 + k * _C
        pltpu.async_copy(agg_sh.at[pl.ds(off, _C)],
                         out_hbm.at[pl.ds(c * _NPAD + off, _C)], semw)
    for k in range(5):
        pltpu.make_async_copy(agg_sh.at[pl.ds(0, _C)],
                              out_hbm.at[pl.ds(0, _C)], semw).wait()


def _conv_call(xn_pad, srcm, dstm):
    mesh = plsc.VectorSubcoreMesh(core_axis_name="c", subcore_axis_name="s")
    return pl.kernel(
        _conv_body,
        out_type=jax.ShapeDtypeStruct((2 * _NPAD, _D), _f32),
        mesh=mesh,
        scratch_types=[
            pltpu.VMEM((_C,), jnp.int32),
            pltpu.VMEM((_C,), jnp.int32),
            pltpu.VMEM((80, _C), jnp.int32),
            pltpu.VMEM((_C, _D), _f32),
            pltpu.VMEM((_C, _D), _f32),
            pltpu.VMEM_SHARED((_NPAD, _D), _f32),
            pltpu.SemaphoreType.DMA,
            pltpu.SemaphoreType.DMA,
            pltpu.SemaphoreType.DMA,
            pltpu.SemaphoreType.DMA,
            pltpu.SemaphoreType.DMA,
            pltpu.SemaphoreType.DMA,
            pltpu.SemaphoreType.DMA,
        ],
    )(xn_pad, srcm, dstm)


def _pair_body(h_hbm, p0_hbm, p1_hbm, n0_hbm, n1_hbm, oP, oN,
               iaA, ibA, iaB, ibB, uA, vA, uB, vB,
               semIA, semIB, semGA, semGB, semWA, semWB):
    c = lax.axis_index("c")
    s = lax.axis_index("s")
    wid = s * _NC + c

    def vadd(u, v):
        def addrow(i, _2):
            for k in range(8):
                sl = pl.ds(k * 16, 16)
                u[i, sl] = u[i, sl] + v[i, sl]
            return _2
        lax.fori_loop(0, _C, addrow, 0)

    def run(a_hbm, b_hbm, out_hbm, t):
        pltpu.async_copy(a_hbm.at[pl.ds(t * 6400, _C)], iaA, semIA)
        pltpu.async_copy(b_hbm.at[pl.ds(t * 6400, _C)], ibA, semIA)
        pltpu.async_copy(a_hbm.at[pl.ds(t * 6400 + _C, _C)], iaB, semIB)
        pltpu.async_copy(b_hbm.at[pl.ds(t * 6400 + _C, _C)], ibB, semIB)

        def grp(g, carry):
            pltpu.make_async_copy(a_hbm.at[pl.ds(0, _C)], iaA, semIA).wait()
            pltpu.make_async_copy(a_hbm.at[pl.ds(0, _C)], ibA, semIA).wait()
            pltpu.async_copy(h_hbm.at[iaA], uA, semGA)
            pltpu.async_copy(h_hbm.at[ibA], vA, semGA)
            pltpu.make_async_copy(a_hbm.at[pl.ds(0, _C)], iaB, semIB).wait()
            pltpu.make_async_copy(a_hbm.at[pl.ds(0, _C)], ibB, semIB).wait()
            pltpu.async_copy(h_hbm.at[iaB], uB, semGB)
            pltpu.async_copy(h_hbm.at[ibB], vB, semGB)

            pltpu.make_async_copy(h_hbm.at[iaA], uA, semGA).wait()
            pltpu.make_async_copy(h_hbm.at[iaA], vA, semGA).wait()
            vadd(uA, vA)
            pltpu.async_copy(uA, out_hbm.at[pl.ds(t * 6400 + 2 * g * _C, _C)],
                             semWA)

            @pl.when(g < 24)
            def _pfA():
                base = t * 6400 + (2 * g + 2) * _C
                pltpu.async_copy(a_hbm.at[pl.ds(base, _C)], iaA, semIA)
                pltpu.async_copy(b_hbm.at[pl.ds(base, _C)], ibA, semIA)

            pltpu.make_async_copy(h_hbm.at[iaB], uB, semGB).wait()
            pltpu.make_async_copy(h_hbm.at[iaB], vB, semGB).wait()
            vadd(uB, vB)
            pltpu.async_copy(uB,
                             out_hbm.at[pl.ds(t * 6400 + (2 * g + 1) * _C, _C)],
                             semWB)

            @pl.when(g < 24)
            def _pfB():
                base = t * 6400 + (2 * g + 3) * _C
                pltpu.async_copy(a_hbm.at[pl.ds(base, _C)], iaB, semIB)
                pltpu.async_copy(b_hbm.at[pl.ds(base, _C)], ibB, semIB)

            pltpu.make_async_copy(uA, out_hbm.at[pl.ds(0, _C)], semWA).wait()
            pltpu.make_async_copy(uB, out_hbm.at[pl.ds(0, _C)], semWB).wait()
            return carry

        lax.fori_loop(0, 25, grp, 0)

    @pl.when(wid < 16)
    def _():
        run(p0_hbm, p1_hbm, oP, wid)

    @pl.when(wid >= 16)
    def _():
        run(n0_hbm, n1_hbm, oN, wid - 16)


def _pair_call(h2_pad, p0, p1, n0, n1):
    mesh = plsc.VectorSubcoreMesh(core_axis_name="c", subcore_axis_name="s")
    return pl.kernel(
        _pair_body,
        out_type=(jax.ShapeDtypeStruct((_EPPAD, _D), _f32),
                  jax.ShapeDtypeStruct((_EPPAD, _D), _f32)),
        mesh=mesh,
        scratch_types=[
            pltpu.VMEM((_C,), jnp.int32),
            pltpu.VMEM((_C,), jnp.int32),
            pltpu.VMEM((_C,), jnp.int32),
            pltpu.VMEM((_C,), jnp.int32),
            pltpu.VMEM((_C, _D), _f32),
            pltpu.VMEM((_C, _D), _f32),
            pltpu.VMEM((_C, _D), _f32),
            pltpu.VMEM((_C, _D), _f32),
            pltpu.SemaphoreType.DMA,
            pltpu.SemaphoreType.DMA,
            pltpu.SemaphoreType.DMA,
            pltpu.SemaphoreType.DMA,
            pltpu.SemaphoreType.DMA,
            pltpu.SemaphoreType.DMA,
        ],
    )(h2_pad, p0, p1, n0, n1)


def _mm_body(x_ref, w_ref, b_ref, o_ref, *, relu):
    y = jnp.dot(x_ref[...], w_ref[...], preferred_element_type=_f32)
    y = y + b_ref[...]
    o_ref[...] = jnp.maximum(y, 0.0) if relu else y


def _mm(xmat, W, b, relu):
    rows = xmat.shape[0]
    blk = 512
    return pl.pallas_call(
        functools.partial(_mm_body, relu=relu),
        grid=(rows // blk,),
        in_specs=[
            pl.BlockSpec((blk, _D), lambda r: (r, 0)),
            pl.BlockSpec((_D, _D), lambda r: (0, 0)),
            pl.BlockSpec((1, _D), lambda r: (0, 0)),
        ],
        out_specs=pl.BlockSpec((blk, _D), lambda r: (r, 0)),
        out_shape=jax.ShapeDtypeStruct((rows, _D), _f32),
    )(xmat, W, b.reshape(1, _D))


def _branch_body(tinv_ref, np_ref, noise_ref, u_ref, Wm_ref, bm_ref,
                 Wl_ref, bl_ref, Wq_ref, bq_ref, Rep_ref, F_ref,
                 Wd1_ref, bd1_ref, WdX_ref, bdX_ref, Wa_ref, ba_ref,
                 P8_ref, mean_o, logstd_o, aq_o, X_o):
    npb = np_ref[...]
    mean = jnp.dot(npb, Wm_ref[...], preferred_element_type=_f32) + bm_ref[...]
    mean_o[...] = mean
    logstd = jnp.dot(npb, Wl_ref[...], preferred_element_type=_f32) + bl_ref[...]
    logstd_o[...] = logstd
    q = jnp.dot(npb, Wq_ref[...], preferred_element_type=_f32) + bq_ref[...]

    u = u_ref[...]
    g = -jnp.log(-jnp.log(u + _EPSV) + _EPSV)
    col = lax.broadcasted_iota(jnp.int32, q.shape, 1)
    logits = (q + jnp.pad(g, ((0, 0), (0, _D - _CAT)))) * tinv_ref[0, 0]
    logits = jnp.where(col < _CAT, logits, -1e30)
    logits = logits - jnp.max(logits, axis=-1, keepdims=True)
    ez = jnp.exp(logits)
    z = ez / jnp.sum(ez, axis=-1, keepdims=True)

    nz = noise_ref[...] * jnp.exp(logstd) + mean
    zbig = jnp.dot(z, Rep_ref[...], preferred_element_type=_f32)
    m = jnp.dot(zbig * nz, F_ref[...], preferred_element_type=_f32)

    x1 = jnp.dot(m, Wd1_ref[...], preferred_element_type=_f32) + bd1_ref[...]
    x1 = jnp.maximum(x1, 0.0)
    x2 = jnp.dot(x1, WdX_ref[...], preferred_element_type=_f32) + bdX_ref[...]
    X_o[...] = jnp.maximum(x2, 0.0)

    al = jnp.dot(m, Wa_ref[...], preferred_element_type=_f32) + ba_ref[...]
    al = al - jnp.max(al, axis=-1, keepdims=True)
    ea = jnp.exp(al)
    a = ea / jnp.sum(ea, axis=-1, keepdims=True)
    aq_o[...] = a + jnp.dot(q, P8_ref[...], preferred_element_type=_f32)


def _branch_call(tinv, npemb_pad, noise, u, weights):
    blk = 400
    grid = (_EP // blk,)
    full = lambda shp: pl.BlockSpec(shp, lambda r: (0, 0))
    return pl.pallas_call(
        _branch_body,
        grid=grid,
        in_specs=[
            pl.BlockSpec(memory_space=pltpu.SMEM),
            pl.BlockSpec((blk, _D), lambda r: (r, 0)),
            pl.BlockSpec((blk, _H2), lambda r: (r, 0)),
            pl.BlockSpec((blk, _CAT), lambda r: (r, 0)),
            full((_D, _H2)), full((1, _H2)),
            full((_D, _H2)), full((1, _H2)),
            full((_D, _D)), full((1, _D)),
            full((_D, _H2)),
            full((_H2, _D)),
            full((_D, _D)), full((1, _D)),
            full((_D, _D)), full((1, _D)),
            full((_D, _D)), full((1, _D)),
            full((_D, _D)),
        ],
        out_specs=[
            pl.BlockSpec((blk, _H2), lambda r: (r, 0)),
            pl.BlockSpec((blk, _H2), lambda r: (r, 0)),
            pl.BlockSpec((blk, _D), lambda r: (r, 0)),
            pl.BlockSpec((blk, _D), lambda r: (r, 0)),
        ],
        out_shape=[
            jax.ShapeDtypeStruct((_EP, _H2), _f32),
            jax.ShapeDtypeStruct((_EP, _H2), _f32),
            jax.ShapeDtypeStruct((_EP, _D), _f32),
            jax.ShapeDtypeStruct((_EP, _D), _f32),
        ],
    )(tinv, npemb_pad, noise, u, *weights)


_USE_SC_DEG = True
_USE_SC_CONV = True
_USE_SC_PAIR = True


def _deg_spec(idx_all):
    co = jnp.zeros((_NPAD,), _f32).at[idx_all[:_EPAD]].add(1.0)
    ci = jnp.zeros((_NPAD,), _f32).at[idx_all[_EPAD:]].add(1.0)
    return jnp.concatenate([co, ci])[:, None] * jnp.ones((1, _D), _f32)


def _conv_spec(xn_pad, srcm, dstm):
    agg = jax.ops.segment_sum(xn_pad[srcm.reshape(-1)], dstm.reshape(-1),
                              num_segments=_NPAD)
    return jnp.zeros((2 * _NPAD, _D), _f32).at[:_NPAD].set(agg)


def _pair_spec(h2, p0, p1, n0, n1):
    return h2[p0] + h2[p1], h2[n0] + h2[n1]


def kernel(x, edge_index, pos_edge_index, neg_edge_index, temp,
           W0, b0, W1, b1, Wm, bm, Wl, bl, Wq, bq,
           Wd1, bd1, WdX, bdX, Wa, ba):
    ei_pad = jnp.pad(edge_index, ((0, 0), (0, _EPAD - _E)),
                     constant_values=_N)
    src_pad = ei_pad[0]
    dst_pad = ei_pad[1]

    idx_all = jnp.concatenate([src_pad, dst_pad])
    cnt = (_deg_call if _USE_SC_DEG else _deg_spec)(idx_all)
    deg_out = cnt[:_N, 0]
    deg_in = cnt[_NPAD:_NPAD + _N, 0]
    norm_out = jnp.where(deg_out > 0, deg_out ** -0.5, 0.0)
    norm_in = jnp.where(deg_in > 0, deg_in ** -0.5, 0.0)
    norm_out_pad = jnp.pad(norm_out, (0, _NPAD - _N))[:, None]
    norm_in_pad = jnp.pad(norm_in, (0, _NPAD - _N))[:, None]

    xn = jnp.pad(x * norm_out[:, None], ((0, _NPAD - _N), (0, 0)))
    srcm = src_pad.reshape(2560, _C)
    dstm = dst_pad.reshape(2560, _C)
    aggp = (_conv_call if _USE_SC_CONV else _conv_spec)(xn, srcm, dstm)
    agg = (aggp[:_NPAD] + aggp[_NPAD:]) * norm_in_pad
    h = _mm(agg, W0, b0, relu=True)
    hn = h * norm_out_pad

    aggp2 = (_conv_call if _USE_SC_CONV else _conv_spec)(hn, srcm, dstm)
    agg2 = (aggp2[:_NPAD] + aggp2[_NPAD:]) * norm_in_pad
    h2 = _mm(agg2, W1, b1, relu=False)

    posi = jnp.pad(pos_edge_index, ((0, 0), (0, _EPPAD - _EP)))
    negi = jnp.pad(neg_edge_index, ((0, 0), (0, _EPPAD - _EP)))
    npP, npN = (_pair_call if _USE_SC_PAIR else _pair_spec)(h2, posi[0], posi[1], negi[0], negi[1])

    noiseP = jax.random.normal(jax.random.key(42), (_EP, _H2), _f32)
    uP = jax.random.uniform(jax.random.key(43), (_EP, _CAT), _f32)
    noiseN = jax.random.normal(jax.random.key(44), (_EP, _H2), _f32)
    uN = jax.random.uniform(jax.random.key(45), (_EP, _CAT), _f32)
    Wqp = jnp.zeros((_D, _D), _f32).at[:, :_CAT].set(Wq)
    bqp = jnp.zeros((1, _D), _f32).at[0, :_CAT].set(bq)
    r_ = jnp.arange(_D)[:, None]
    c_ = jnp.arange(_H2)[None, :]
    Rep = (c_ // 32 == r_).astype(_f32)
    i_ = jnp.arange(_H2)[:, None]
    j_ = jnp.arange(_D)[None, :]
    Fm = (i_ % 32 == j_).astype(_f32)
    Wd1p = jnp.zeros((_D, _D), _f32).at[:32, :64].set(Wd1)
    bd1p = jnp.zeros((1, _D), _f32).at[0, :64].set(bd1)
    WdXp = jnp.zeros((_D, _D), _f32).at[:64, :].set(WdX)
    bdXp = bdX.reshape(1, _D)
    Wap = jnp.zeros((_D, _D), _f32).at[:32, :_CAT].set(Wa)
    bap = jnp.full((1, _D), -1e30, _f32).at[0, :_CAT].set(ba)
    P8 = (jnp.arange(_D)[:, None] + 8 == jnp.arange(_D)[None, :]).astype(_f32)
    P8 = P8.at[_CAT:, :].set(0.0)

    weights = (Wm, bm.reshape(1, _H2), Wl, bl.reshape(1, _H2),
               Wqp, bqp, Rep, Fm, Wd1p, bd1p, WdXp, bdXp, Wap, bap, P8)
    tinv = (1.0 / jnp.asarray(temp, _f32)).reshape(1, 1)

    pos_mean, pos_logstd, posAQ, posX = _branch_call(
        tinv, npP, noiseP, uP, weights)
    neg_mean, neg_logstd, negAQ, negX = _branch_call(
        tinv, npN, noiseN, uN, weights)

    return (posAQ[:, :_CAT], negAQ[:, :_CAT], posX, negX,
            pos_mean, neg_mean, pos_logstd, neg_logstd,
            posAQ[:, _CAT:2 * _CAT], negAQ[:, _CAT:2 * _CAT])
